# baseline (speedup 1.0000x reference)
import jax
import jax.numpy as jnp
from jax.experimental import pallas as pl
from jax.experimental.pallas import tpu as pltpu

_B = 256
_G = 20000
_P = 500
_KBLK = 2048
_NBLK = -(-_G // _KBLK)
_MIN_SIZE = 5.0


def _pcl_body(et_ref, pt_ref, m_ref, out_ref, acc_ref, size_ref):
    k = pl.program_id(0)

    @pl.when(k == 0)
    def _init():
        acc_ref[...] = jnp.zeros_like(acc_ref)
        size_ref[...] = jnp.zeros_like(size_ref)

    limit = _G - k * _KBLK
    gene_rows = jax.lax.broadcasted_iota(jnp.int32, (_KBLK, _B), 0)
    gene_lanes = jax.lax.broadcasted_iota(jnp.int32, (_P, _KBLK), 1)
    d = jnp.where(gene_rows < limit, pt_ref[...] - et_ref[...], 0.0)
    m = jnp.where(gene_lanes < limit, m_ref[...], 0.0)

    acc_ref[...] += jax.lax.dot_general(
        m, d, (((1,), (0,)), ((), ())),
        preferred_element_type=jnp.float32)
    ones = jnp.ones((_KBLK, 8), jnp.float32)
    size_ref[...] += jax.lax.dot_general(
        m, ones, (((1,), (0,)), ((), ())),
        preferred_element_type=jnp.float32)

    @pl.when(k == _NBLK - 1)
    def _finalize():
        sizes = size_ref[:, 0:1]
        safe = jnp.maximum(sizes, 1.0)
        mean_diff = acc_ref[...] / safe
        mse = jnp.mean(mean_diff * mean_diff, axis=1, keepdims=True)
        valid = (sizes >= _MIN_SIZE).astype(jnp.float32)
        n_valid = jnp.sum(valid, axis=(0, 1), keepdims=True)
        total = jnp.sum(mse * valid, axis=(0, 1), keepdims=True)
        out_ref[...] = jnp.where(
            n_valid > 0.0, total / jnp.maximum(n_valid, 1.0), 0.0)


def kernel(expression, predicted, pathway_gene_matrix):
    out = pl.pallas_call(
        _pcl_body,
        grid=(_NBLK,),
        in_specs=[
            pl.BlockSpec((_KBLK, _B), lambda k: (k, 0)),
            pl.BlockSpec((_KBLK, _B), lambda k: (k, 0)),
            pl.BlockSpec((_P, _KBLK), lambda k: (0, k)),
        ],
        out_specs=pl.BlockSpec((1, 1), lambda k: (0, 0)),
        out_shape=jax.ShapeDtypeStruct((1, 1), jnp.float32),
        scratch_shapes=[
            pltpu.VMEM((_P, _B), jnp.float32),
            pltpu.VMEM((_P, 8), jnp.float32),
        ],
        compiler_params=pltpu.CompilerParams(
            dimension_semantics=("arbitrary",),
        ),
    )(expression.T, predicted.T, pathway_gene_matrix)
    return out[0, 0]
